# Initial kernel scaffold; baseline (speedup 1.0000x reference)
import jax
import jax.numpy as jnp
from jax.experimental import pallas as pl


def kernel(features, rbf_expansion, neighbor_list, W1, b1, W2, b2):
    raise NotImplementedError("write your pallas kernel here")


# speedup vs baseline: 2.4732x; 2.4732x over previous
import functools

import jax
import jax.numpy as jnp
from jax import lax
from jax.experimental import pallas as pl
from jax.experimental.pallas import tpu as pltpu
from jax.experimental.pallas import tpu_sc as plsc

B, NBEADS, NNBR = 8, 1024, 32
NG, NF = 64, 128
ROWS = B * NBEADS * NNBR
NW = 32
RPW = ROWS // NW
CH = 256
NCH = RPW // CH
LANES = 16

_LN2 = 0.6931471805599453


def _sc_gather(features_flat, nl_flat):
    mesh = plsc.VectorSubcoreMesh(core_axis_name="c", subcore_axis_name="s")

    @functools.partial(
        pl.kernel,
        out_type=jax.ShapeDtypeStruct((ROWS, NF), jnp.float32),
        mesh=mesh,
        scratch_types=[
            pltpu.VMEM((RPW,), jnp.int32),
            pltpu.VMEM((CH, NF), jnp.float32),
            pltpu.SemaphoreType.DMA,
        ],
    )
    def gather_kernel(feat_hbm, nl_hbm, out_hbm, idx_v, buf, gsem):
        wid = lax.axis_index("s") * 2 + lax.axis_index("c")
        b = wid // 4
        q = wid % 4
        pltpu.sync_copy(nl_hbm.at[pl.ds(q * RPW, RPW)], idx_v)
        boff = lax.broadcast(b * NBEADS, (LANES,))

        def add_off(j, _):
            idx_v[pl.ds(j * LANES, LANES)] = idx_v[pl.ds(j * LANES, LANES)] + boff
            return 0

        lax.fori_loop(0, RPW // LANES, add_off, 0)

        base = wid * RPW

        def do_chunk(c, _):
            pltpu.async_copy(
                feat_hbm.at[idx_v.at[pl.ds(c * CH, CH)]], buf, gsem
            ).wait()
            pltpu.sync_copy(buf, out_hbm.at[pl.ds(base + c * CH, CH)])
            return 0

        lax.fori_loop(0, NCH, do_chunk, 0)

    return gather_kernel(features_flat, nl_flat)


ROWBLK = 4096
BEADBLK = ROWBLK // NNBR


def _tc_body(rbf_ref, nf_ref, w1_ref, b1_ref, w2_ref, b2_ref, out_ref):
    x = rbf_ref[...]
    h = jnp.dot(x, w1_ref[...], preferred_element_type=jnp.float32)
    h = h + b1_ref[...]
    h = jnp.maximum(h, 0.0) + jnp.log(1.0 + jnp.exp(-jnp.abs(h))) - _LN2
    f = jnp.dot(h, w2_ref[...], preferred_element_type=jnp.float32)
    f = f + b2_ref[...]
    p = f * nf_ref[...]
    out_ref[...] = jnp.sum(p.reshape(BEADBLK, NNBR, NF), axis=1)


def _tc_compute(rbf_flat, nf_flat, W1, b1, W2, b2):
    grid = (ROWS // ROWBLK,)
    return pl.pallas_call(
        _tc_body,
        grid=grid,
        in_specs=[
            pl.BlockSpec((ROWBLK, NG), lambda i: (i, 0)),
            pl.BlockSpec((ROWBLK, NF), lambda i: (i, 0)),
            pl.BlockSpec((NG, NF), lambda i: (0, 0)),
            pl.BlockSpec((1, NF), lambda i: (0, 0)),
            pl.BlockSpec((NF, NF), lambda i: (0, 0)),
            pl.BlockSpec((1, NF), lambda i: (0, 0)),
        ],
        out_specs=pl.BlockSpec((BEADBLK, NF), lambda i: (i, 0)),
        out_shape=jax.ShapeDtypeStruct((B * NBEADS, NF), jnp.float32),
    )(rbf_flat, nf_flat, W1, b1, W2, b2)


def kernel(features, rbf_expansion, neighbor_list, W1, b1, W2, b2):
    features_flat = features.reshape(B * NBEADS, NF)
    nl_flat = neighbor_list.reshape(ROWS // B)
    nf_flat = _sc_gather(features_flat, nl_flat)
    agg = _tc_compute(
        rbf_expansion.reshape(ROWS, NG),
        nf_flat,
        W1,
        b1.reshape(1, NF),
        W2,
        b2.reshape(1, NF),
    )
    return agg.reshape(B, NBEADS, NF)
